# Initial kernel scaffold; baseline (speedup 1.0000x reference)
import functools

import jax
import jax.numpy as jnp
from jax import lax
from jax.experimental import pallas as pl
from jax.experimental.pallas import tpu as pltpu
from jax.experimental.pallas import tpu_sc as plsc

_NUM_CLASSES = 1000
_PAD = 1024
_BATCH = 16384
_FEAT = 128
_NC = 2
_NS = 16
_ROWS_PER_TILE = _BATCH // (_NC * _NS)
_CHUNK = 128
_NCHUNK = _ROWS_PER_TILE // _CHUNK
_INIT_ROWS = _PAD // _NS
_CNT_W = 16


def _sc_segment_sums(x, t2, zacc, zcnt, ones_rows):
    mesh = plsc.VectorSubcoreMesh(core_axis_name="c", subcore_axis_name="s")

    @functools.partial(
        pl.kernel,
        out_type=(
            jax.ShapeDtypeStruct((_NC, _PAD, _FEAT), jnp.float32),
            jax.ShapeDtypeStruct((_NC, _PAD, _CNT_W), jnp.float32),
        ),
        mesh=mesh,
        scratch_types=[
            pltpu.VMEM_SHARED((_PAD, _FEAT), jnp.float32),
            pltpu.VMEM_SHARED((_PAD, _CNT_W), jnp.float32),
            pltpu.VMEM((_NCHUNK, _CHUNK), jnp.int32),
            pltpu.VMEM((2, _CHUNK, _FEAT), jnp.float32),
            pltpu.VMEM((_CHUNK, _CNT_W), jnp.float32),
            pltpu.SemaphoreType.DMA,
        ],
    )
    def k(x_hbm, t_hbm, zacc_hbm, zcnt_hbm, ones_hbm, out_s, out_c,
          acc, cnt, idx_v, rows_v, ones_v, sem):
        core = lax.axis_index("c")
        sub = lax.axis_index("s")
        r0 = sub * _INIT_ROWS
        pltpu.sync_copy(zacc_hbm.at[pl.ds(r0, _INIT_ROWS)],
                        acc.at[pl.ds(r0, _INIT_ROWS)])
        pltpu.sync_copy(zcnt_hbm.at[pl.ds(r0, _INIT_ROWS)],
                        cnt.at[pl.ds(r0, _INIT_ROWS)])
        trow = core * (_NS * _NCHUNK) + sub * _NCHUNK
        pltpu.sync_copy(t_hbm.at[pl.ds(trow, _NCHUNK)], idx_v)
        pltpu.sync_copy(ones_hbm, ones_v)
        row_base = core * (_BATCH // _NC) + sub * _ROWS_PER_TILE
        plsc.subcore_barrier()

        pltpu.async_copy(x_hbm.at[pl.ds(row_base, _CHUNK)], rows_v.at[0],
                         sem).wait()
        for j in range(_NCHUNK):
            if j + 1 < _NCHUNK:
                nxt = pltpu.async_copy(
                    x_hbm.at[pl.ds(row_base + (j + 1) * _CHUNK, _CHUNK)],
                    rows_v.at[(j + 1) % 2], sem)
            pltpu.sync_copy(rows_v.at[j % 2], acc.at[idx_v.at[j]], add=True)
            pltpu.sync_copy(ones_v, cnt.at[idx_v.at[j]], add=True)
            if j + 1 < _NCHUNK:
                nxt.wait()

        plsc.subcore_barrier()
        pltpu.sync_copy(acc.at[pl.ds(r0, _INIT_ROWS)],
                        out_s.at[core, pl.ds(r0, _INIT_ROWS)])
        pltpu.sync_copy(cnt.at[pl.ds(r0, _INIT_ROWS)],
                        out_c.at[core, pl.ds(r0, _INIT_ROWS)])

    return k(x, t2, zacc, zcnt, ones_rows)


def _sumsq(x):
    def body(x_ref, o_ref):
        @pl.when(pl.program_id(0) == 0)
        def _():
            o_ref[...] = jnp.zeros_like(o_ref)
        xb = x_ref[...]
        o_ref[...] += jnp.sum(xb * xb, axis=0, keepdims=True)

    return pl.pallas_call(
        body,
        grid=(_BATCH // 2048,),
        in_specs=[pl.BlockSpec((2048, _FEAT), lambda i: (i, 0))],
        out_specs=pl.BlockSpec((1, _FEAT), lambda i: (0, 0)),
        out_shape=jax.ShapeDtypeStruct((1, _FEAT), jnp.float32),
    )(x)


def _finalize(sums, cnts, ss):
    def body(s_ref, c_ref, ss_ref, o_ref):
        s = s_ref[0] + s_ref[1]
        c = c_ref[0] + c_ref[1]
        n = c[:, 0:1]
        sq = jnp.sum(s * s, axis=1, keepdims=True)
        nz = n > 0.0
        term = jnp.sum(jnp.where(nz, sq / jnp.where(nz, n, 1.0), 0.0))
        n_ids = jnp.sum(jnp.where(nz, 1.0, 0.0))
        loss = 0.5 * (jnp.sum(ss_ref[...]) - term)
        o_ref[...] = jnp.where(n_ids == float(_BATCH), 0.0, loss).reshape(1, 1)

    return pl.pallas_call(
        body,
        out_shape=jax.ShapeDtypeStruct((1, 1), jnp.float32),
    )(sums, cnts, ss)


def kernel(inputs, targets):
    t2 = targets.reshape(_BATCH // _CHUNK, _CHUNK).astype(jnp.int32)
    zacc = jnp.zeros((_PAD, _FEAT), jnp.float32)
    zcnt = jnp.zeros((_PAD, _CNT_W), jnp.float32)
    ones_rows = jnp.zeros((_CHUNK, _CNT_W), jnp.float32).at[:, 0].set(1.0)
    sums, cnts = _sc_segment_sums(inputs, t2, zacc, zcnt, ones_rows)
    ss = _sumsq(inputs)
    out = _finalize(sums, cnts, ss)
    return out[0, 0]


# speedup vs baseline: 5.6521x; 5.6521x over previous
import functools

import jax
import jax.numpy as jnp
from jax import lax
from jax.experimental import pallas as pl
from jax.experimental.pallas import tpu as pltpu
from jax.experimental.pallas import tpu_sc as plsc

_NUM_CLASSES = 1000
_PAD = 1024
_BATCH = 16384
_FEAT = 128
_NC = 2
_NS = 16
_ROWS_PER_TILE = _BATCH // (_NC * _NS)
_CHUNK = 128
_NCHUNK = _ROWS_PER_TILE // _CHUNK
_INIT_ROWS = _PAD // _NS


def _sc_segment_sums(x, t2, zacc, ones_rows):
    mesh = plsc.VectorSubcoreMesh(core_axis_name="c", subcore_axis_name="s")

    @functools.partial(
        pl.kernel,
        out_type=(
            jax.ShapeDtypeStruct((_NC, _PAD, _FEAT), jnp.float32),
            jax.ShapeDtypeStruct((_NC, _PAD, _FEAT), jnp.float32),
        ),
        mesh=mesh,
        scratch_types=[
            pltpu.VMEM_SHARED((_PAD, _FEAT), jnp.float32),
            pltpu.VMEM_SHARED((_PAD, _FEAT), jnp.float32),
            pltpu.VMEM((_NCHUNK, _CHUNK), jnp.int32),
            pltpu.VMEM((2, _CHUNK, _FEAT), jnp.float32),
            pltpu.VMEM((_CHUNK, _FEAT), jnp.float32),
            pltpu.SemaphoreType.DMA,
        ],
    )
    def k(x_hbm, t_hbm, zacc_hbm, ones_hbm, out_s, out_c,
          acc, cnt, idx_v, rows_v, ones_v, sem):
        core = lax.axis_index("c")
        sub = lax.axis_index("s")
        r0 = sub * _INIT_ROWS
        pltpu.sync_copy(zacc_hbm.at[pl.ds(r0, _INIT_ROWS)],
                        acc.at[pl.ds(r0, _INIT_ROWS)])
        pltpu.sync_copy(zacc_hbm.at[pl.ds(r0, _INIT_ROWS)],
                        cnt.at[pl.ds(r0, _INIT_ROWS)])
        trow = core * (_NS * _NCHUNK) + sub * _NCHUNK
        pltpu.sync_copy(t_hbm.at[pl.ds(trow, _NCHUNK)], idx_v)
        pltpu.sync_copy(ones_hbm, ones_v)
        row_base = core * (_BATCH // _NC) + sub * _ROWS_PER_TILE
        plsc.subcore_barrier()

        pltpu.async_copy(x_hbm.at[pl.ds(row_base, _CHUNK)], rows_v.at[0],
                         sem).wait()
        for j in range(_NCHUNK):
            if j + 1 < _NCHUNK:
                nxt = pltpu.async_copy(
                    x_hbm.at[pl.ds(row_base + (j + 1) * _CHUNK, _CHUNK)],
                    rows_v.at[(j + 1) % 2], sem)
            pltpu.sync_copy(rows_v.at[j % 2], acc.at[idx_v.at[j]], add=True)
            pltpu.sync_copy(ones_v, cnt.at[idx_v.at[j]], add=True)
            if j + 1 < _NCHUNK:
                nxt.wait()

        plsc.subcore_barrier()
        pltpu.sync_copy(acc.at[pl.ds(r0, _INIT_ROWS)],
                        out_s.at[core, pl.ds(r0, _INIT_ROWS)])
        pltpu.sync_copy(cnt.at[pl.ds(r0, _INIT_ROWS)],
                        out_c.at[core, pl.ds(r0, _INIT_ROWS)])

    return k(x, t2, zacc, ones_rows)


def _sumsq(x):
    def body(x_ref, o_ref):
        @pl.when(pl.program_id(0) == 0)
        def _():
            o_ref[...] = jnp.zeros_like(o_ref)
        xb = x_ref[...]
        o_ref[...] += jnp.sum(xb * xb, axis=0, keepdims=True)

    return pl.pallas_call(
        body,
        grid=(_BATCH // 2048,),
        in_specs=[pl.BlockSpec((2048, _FEAT), lambda i: (i, 0))],
        out_specs=pl.BlockSpec((1, _FEAT), lambda i: (0, 0)),
        out_shape=jax.ShapeDtypeStruct((1, _FEAT), jnp.float32),
    )(x)


def _finalize(sums, cnts, ss):
    def body(s_ref, c_ref, ss_ref, o_ref):
        s = s_ref[0] + s_ref[1]
        n = c_ref[0, :, 0:1] + c_ref[1, :, 0:1]
        sq = jnp.sum(s * s, axis=1, keepdims=True)
        nz = n > 0.0
        term = jnp.sum(jnp.where(nz, sq / jnp.where(nz, n, 1.0), 0.0))
        n_ids = jnp.sum(jnp.where(nz, 1.0, 0.0))
        loss = 0.5 * (jnp.sum(ss_ref[...]) - term)
        o_ref[...] = jnp.where(n_ids == float(_BATCH), 0.0, loss).reshape(1, 1)

    return pl.pallas_call(
        body,
        out_shape=jax.ShapeDtypeStruct((1, 1), jnp.float32),
    )(sums, cnts, ss)


def kernel(inputs, targets):
    t2 = targets.reshape(_BATCH // _CHUNK, _CHUNK).astype(jnp.int32)
    zacc = jnp.zeros((_PAD, _FEAT), jnp.float32)
    ones_rows = jnp.ones((_CHUNK, _FEAT), jnp.float32)
    sums, cnts = _sc_segment_sums(inputs, t2, zacc, ones_rows)
    ss = _sumsq(inputs)
    out = _finalize(sums, cnts, ss)
    return out[0, 0]
